# baseline (speedup 1.0000x reference)
import jax
import jax.numpy as jnp
from jax import lax
from jax.experimental import pallas as pl
from jax.experimental.pallas import tpu as pltpu
from jax.experimental.pallas import tpu_sc as plsc

M = 100000
B = 16384
D = 128
MOM = 0.8

NC, NS, L = 2, 16, 16
NW = NC * NS
W = 400
TOTWIN = M // W
NBASE = TOTWIN // NW
EXTRA = TOTWIN % NW
MAXWIN = NBASE + 1
IDXC = 2048
NCHUNK = B // IDXC
KPW = (W + L - 1) // L
SSTR = KPW * L
SLOT_SZ = MAXWIN * SSTR + L
GC = 128
GROWS = (MAXWIN * W + GC - 1) // GC


_GATHER_DNUMS = lax.GatherDimensionNumbers(
    offset_dims=(), collapsed_slice_dims=(0,), start_index_map=(0,))


def _gather16(x, idx):
    return lax.gather(x, idx[:, None], _GATHER_DNUMS, slice_sizes=(1,),
                      mode=lax.GatherScatterMode.PROMISE_IN_BOUNDS)


def _prefix_incl(x, lanes):
    c = x
    for s in (1, 2, 4, 8):
        sh = _gather16(c, jnp.maximum(lanes - s, 0))
        c = c + jnp.where(lanes >= s, sh, jnp.int32(0))
    return c


def _body(mem_hbm, val_hbm, idx_hbm, out_hbm,
          ibuf0, ibuf1, rowl, posl, slot, grow, gpos, wbuf0, wbuf1,
          isem0, isem1, lsem0, lsem1, ssem0, ssem1, gsem0, gsem1):
    i32 = jnp.int32
    f32 = jnp.float32
    wid = lax.axis_index("s") * NC + lax.axis_index("c")
    nwin = jnp.where(wid < EXTRA, NBASE + 1, NBASE)
    first_win = wid * (NBASE + 1) - jnp.maximum(0, wid - EXTRA)
    lo = first_win * W
    rng = nwin * W
    lanes = lax.iota(i32, L)
    neg1 = jnp.full((L,), -1, i32)

    def init_slot(i, _):
        slot[pl.ds(i * L, L)] = neg1
        return 0
    lax.fori_loop(0, SLOT_SZ // L, init_slot, 0)

    def process_chunk(c, ibuf):
        def scan_vreg(i, off):
            v = ibuf[pl.ds(i * L, L)]
            m = (v >= lo) & (v < lo + rng)
            n = plsc.all_reduce_population_count(m)[0]

            @pl.when(n > 0)
            def _():
                mi = jnp.where(m, jnp.int32(1), jnp.int32(0))
                csum = _prefix_incl(mi, lanes)
                dest = off + csum - mi
                rl = v - lo
                widx = rl if SSTR == W else (rl // W) * SSTR + rl % W
                pos = c * IDXC + i * L + lanes
                plsc.store_scatter(rowl, [dest], widx, mask=m)
                plsc.store_scatter(posl, [dest], pos, mask=m)
            return off + n
        cnt = lax.fori_loop(0, IDXC // L, scan_vreg, jnp.int32(0))

        def dedup(i, _):
            rv = rowl[pl.ds(i * L, L)]
            pv = posl[pl.ds(i * L, L)]
            for l in range(L):
                mk = (lanes == l) & (i * L + l < cnt)
                plsc.store_scatter(slot, [rv], pv, mask=mk)
            return 0
        lax.fori_loop(0, (cnt + L - 1) // L, dedup, 0)

    def idx_load(c, ibuf, sem):
        pltpu.async_copy(idx_hbm.at[pl.ds(c * IDXC, IDXC)], ibuf, sem)

    def idx_wait(ibuf, sem):
        pltpu.make_async_copy(idx_hbm.at[pl.ds(0, IDXC)], ibuf, sem).wait()

    def load_issue(c, wb, sem):
        pltpu.async_copy(mem_hbm.at[pl.ds(lo + c * W, W)], wb, sem)

    def load_wait(wb, sem):
        pltpu.make_async_copy(mem_hbm.at[pl.ds(0, W)], wb, sem).wait()

    def store_issue(c, wb, sem):
        pltpu.async_copy(wb, out_hbm.at[pl.ds(lo + c * W, W)], sem)

    def store_wait(wb, sem):
        pltpu.make_async_copy(wb, out_hbm.at[pl.ds(0, W)], sem).wait()

    def win_step(c, wb_p, lsem_p, ssem_p, wb_q, lsem_q, ssem_q):
        @pl.when(c < nwin)
        def _():
            load_wait(wb_p, lsem_p)
            store_issue(c, wb_p, ssem_p)

        @pl.when(c + 1 < nwin)
        def _():
            @pl.when(c >= 1)
            def _():
                store_wait(wb_q, ssem_q)
            load_issue(c + 1, wb_q, lsem_q)

    idx_load(0, ibuf0, isem0)
    load_issue(0, wbuf0, lsem0)

    def chunk_pair(t, _):
        c0 = 2 * t
        win_step(c0, wbuf0, lsem0, ssem0, wbuf1, lsem1, ssem1)
        idx_wait(ibuf0, isem0)
        idx_load(c0 + 1, ibuf1, isem1)
        process_chunk(c0, ibuf0)

        win_step(c0 + 1, wbuf1, lsem1, ssem1, wbuf0, lsem0, ssem0)
        idx_wait(ibuf1, isem1)

        @pl.when(t < NCHUNK // 2 - 1)
        def _():
            idx_load(c0 + 2, ibuf0, isem0)
        process_chunk(c0 + 1, ibuf1)
        return 0
    lax.fori_loop(0, NCHUNK // 2, chunk_pair, 0)

    store_wait(wbuf0, ssem0)
    store_wait(wbuf1, ssem1)

    def collect_win(wi, u):
        def collect(k, u):
            rloc = k * L + lanes
            sv = slot[pl.ds(wi * SSTR + k * L, L)]
            m = (rloc < W) & (sv >= 0)
            n = plsc.all_reduce_population_count(m)[0]

            @pl.when(n > 0)
            def _():
                mi = jnp.where(m, jnp.int32(1), jnp.int32(0))
                csum = _prefix_incl(mi, lanes)
                dest = u + csum - mi
                rabs = lo + wi * W + rloc
                plsc.store_scatter(grow, [dest // GC, dest % GC],
                                   rabs, mask=m)
                plsc.store_scatter(gpos, [dest // GC, dest % GC],
                                   sv, mask=m)
            return u + n
        return lax.fori_loop(0, KPW, collect, u)
    u = lax.fori_loop(0, MAXWIN, collect_win, jnp.int32(0))

    @pl.when(u > 0)
    def _():
        r0 = jnp.full((L,), grow[0, pl.ds(0, L)][0], i32)
        p0 = jnp.full((L,), gpos[0, pl.ds(0, L)][0], i32)
        ulim = ((u + GC - 1) // GC) * GC
        for k in range(8):
            dest = u + k * L + lanes
            mk = dest < ulim
            plsc.store_scatter(grow, [dest // GC, dest % GC], r0, mask=mk)
            plsc.store_scatter(gpos, [dest // GC, dest % GC], p0, mask=mk)

        mb = wbuf0.at[pl.ds(0, GC)]
        vb = wbuf1.at[pl.ds(0, GC)]

        def chunk_c(g, _):
            pltpu.async_copy(mem_hbm.at[grow.at[g]], mb, gsem0)
            pltpu.async_copy(val_hbm.at[gpos.at[g]], vb, gsem1)
            pltpu.make_async_copy(mem_hbm.at[pl.ds(0, GC)], mb, gsem0).wait()
            pltpu.make_async_copy(val_hbm.at[pl.ds(0, GC)], vb, gsem1).wait()

            def blend(r, _):
                for t_ in range(D // L):
                    a = wbuf0[r, pl.ds(t_ * L, L)]
                    b = wbuf1[r, pl.ds(t_ * L, L)]
                    wbuf0[r, pl.ds(t_ * L, L)] = (
                        f32(1.0 - MOM) * a + f32(MOM) * b)
                return 0
            lax.fori_loop(0, GC, blend, 0)

            pltpu.async_copy(mb, out_hbm.at[grow.at[g]], gsem0)
            pltpu.make_async_copy(mb, out_hbm.at[pl.ds(0, GC)], gsem0).wait()
            return 0
        lax.fori_loop(0, (u + GC - 1) // GC, chunk_c, 0)


@jax.jit
def kernel(mem, val, idx):
    mesh = plsc.VectorSubcoreMesh(
        core_axis_name="c", subcore_axis_name="s",
        num_cores=NC, num_subcores=NS)
    f = pl.kernel(
        _body,
        out_type=jax.ShapeDtypeStruct((M, D), jnp.float32),
        mesh=mesh,
        compiler_params=pltpu.CompilerParams(needs_layout_passes=False),
        scratch_types=[
            pltpu.VMEM((IDXC,), jnp.int32),
            pltpu.VMEM((IDXC,), jnp.int32),
            pltpu.VMEM((IDXC + L,), jnp.int32),
            pltpu.VMEM((IDXC + L,), jnp.int32),
            pltpu.VMEM((SLOT_SZ,), jnp.int32),
            pltpu.VMEM((GROWS, GC), jnp.int32),
            pltpu.VMEM((GROWS, GC), jnp.int32),
            pltpu.VMEM((W, D), jnp.float32),
            pltpu.VMEM((W, D), jnp.float32),
            pltpu.SemaphoreType.DMA,
            pltpu.SemaphoreType.DMA,
            pltpu.SemaphoreType.DMA,
            pltpu.SemaphoreType.DMA,
            pltpu.SemaphoreType.DMA,
            pltpu.SemaphoreType.DMA,
            pltpu.SemaphoreType.DMA,
            pltpu.SemaphoreType.DMA,
        ],
    )
    return f(mem, val, idx)


# speedup vs baseline: 1.0538x; 1.0538x over previous
import jax
import jax.numpy as jnp
from jax import lax
from jax.experimental import pallas as pl
from jax.experimental.pallas import tpu as pltpu
from jax.experimental.pallas import tpu_sc as plsc

M = 100000
B = 16384
D = 128
MOM = 0.8

NC, NS, L = 2, 16, 16
NW = NC * NS
W = 200
TOTWIN = M // W
NBASE = TOTWIN // NW
EXTRA = TOTWIN % NW
MAXWIN = NBASE + 1
IDXC = 1024
NCHUNK = B // IDXC
KPW = (W + L - 1) // L
SSTR = KPW * L
SLOT_SZ = MAXWIN * SSTR + L
GC = 128
GROWS = (MAXWIN * W + GC - 1) // GC


_GATHER_DNUMS = lax.GatherDimensionNumbers(
    offset_dims=(), collapsed_slice_dims=(0,), start_index_map=(0,))


def _gather16(x, idx):
    return lax.gather(x, idx[:, None], _GATHER_DNUMS, slice_sizes=(1,),
                      mode=lax.GatherScatterMode.PROMISE_IN_BOUNDS)


def _prefix_incl(x, lanes):
    c = x
    for s in (1, 2, 4, 8):
        sh = _gather16(c, jnp.maximum(lanes - s, 0))
        c = c + jnp.where(lanes >= s, sh, jnp.int32(0))
    return c


def _body(mem_hbm, val_hbm, idx_hbm, out_hbm,
          ibuf0, ibuf1, rowl, posl, slot, grow, gpos,
          wbuf0, wbuf1, wbuf2, wbuf3,
          isem0, isem1, lsem0, lsem1, lsem2, lsem3,
          ssem0, ssem1, ssem2, ssem3, gsem0, gsem1):
    i32 = jnp.int32
    f32 = jnp.float32
    wid = lax.axis_index("s") * NC + lax.axis_index("c")
    nwin = jnp.where(wid < EXTRA, NBASE + 1, NBASE)
    first_win = wid * (NBASE + 1) - jnp.maximum(0, wid - EXTRA)
    lo = first_win * W
    rng = nwin * W
    lanes = lax.iota(i32, L)
    neg1 = jnp.full((L,), -1, i32)

    def init_slot(i, _):
        slot[pl.ds(i * L, L)] = neg1
        return 0
    lax.fori_loop(0, SLOT_SZ // L, init_slot, 0)

    def process_chunk(c, ibuf):
        def scan_vreg(i, off):
            v = ibuf[pl.ds(i * L, L)]
            m = (v >= lo) & (v < lo + rng)
            n = plsc.all_reduce_population_count(m)[0]

            @pl.when(n > 0)
            def _():
                mi = jnp.where(m, jnp.int32(1), jnp.int32(0))
                csum = _prefix_incl(mi, lanes)
                dest = off + csum - mi
                rl = v - lo
                widx = (rl // W) * SSTR + rl % W
                pos = c * IDXC + i * L + lanes
                plsc.store_scatter(rowl, [dest], widx, mask=m)
                plsc.store_scatter(posl, [dest], pos, mask=m)
            return off + n
        cnt = lax.fori_loop(0, IDXC // L, scan_vreg, jnp.int32(0))

        def dedup(i, _):
            rv = rowl[pl.ds(i * L, L)]
            pv = posl[pl.ds(i * L, L)]
            for l in range(L):
                mk = (lanes == l) & (i * L + l < cnt)
                plsc.store_scatter(slot, [rv], pv, mask=mk)
            return 0
        lax.fori_loop(0, (cnt + L - 1) // L, dedup, 0)

    def idx_load(c, ibuf, sem):
        pltpu.async_copy(idx_hbm.at[pl.ds(c * IDXC, IDXC)], ibuf, sem)

    def idx_wait(ibuf, sem):
        pltpu.make_async_copy(idx_hbm.at[pl.ds(0, IDXC)], ibuf, sem).wait()

    def load_issue(c, wb, sem):
        pltpu.async_copy(mem_hbm.at[pl.ds(lo + c * W, W)], wb, sem)

    def load_wait(wb, sem):
        pltpu.make_async_copy(mem_hbm.at[pl.ds(0, W)], wb, sem).wait()

    def store_issue(c, wb, sem):
        pltpu.async_copy(wb, out_hbm.at[pl.ds(lo + c * W, W)], sem)

    def store_wait(wb, sem):
        pltpu.make_async_copy(wb, out_hbm.at[pl.ds(0, W)], sem).wait()

    WB = ((wbuf0, lsem0, ssem0), (wbuf1, lsem1, ssem1),
          (wbuf2, lsem2, ssem2), (wbuf3, lsem3, ssem3))

    def win_step(c, s):
        wb_p, lsem_p, ssem_p = WB[s]
        wb_q, lsem_q, ssem_q = WB[(s + 3) % 4]

        @pl.when(c < nwin)
        def _():
            load_wait(wb_p, lsem_p)
            store_issue(c, wb_p, ssem_p)

        @pl.when(c + 3 < nwin)
        def _():
            @pl.when(c >= 1)
            def _():
                store_wait(wb_q, ssem_q)
            load_issue(c + 3, wb_q, lsem_q)

    idx_load(0, ibuf0, isem0)
    load_issue(0, wbuf0, lsem0)
    load_issue(1, wbuf1, lsem1)
    load_issue(2, wbuf2, lsem2)

    def chunk_quad(q, _):
        for s in range(4):
            c = 4 * q + s
            ib, isem = (ibuf0, isem0) if s % 2 == 0 else (ibuf1, isem1)
            nb, nsem = (ibuf1, isem1) if s % 2 == 0 else (ibuf0, isem0)
            win_step(c, s)
            idx_wait(ib, isem)
            if s < 3:
                idx_load(c + 1, nb, nsem)
            else:
                @pl.when(q < NCHUNK // 4 - 1)
                def _():
                    idx_load(c + 1, nb, nsem)
            process_chunk(c, ib)
        return 0
    lax.fori_loop(0, NCHUNK // 4, chunk_quad, 0)

    store_wait(wbuf0, ssem0)
    store_wait(wbuf1, ssem1)
    store_wait(wbuf2, ssem2)
    store_wait(wbuf3, ssem3)

    def collect_win(wi, u):
        def collect(k, u):
            rloc = k * L + lanes
            sv = slot[pl.ds(wi * SSTR + k * L, L)]
            m = (rloc < W) & (sv >= 0)
            n = plsc.all_reduce_population_count(m)[0]

            @pl.when(n > 0)
            def _():
                mi = jnp.where(m, jnp.int32(1), jnp.int32(0))
                csum = _prefix_incl(mi, lanes)
                dest = u + csum - mi
                rabs = lo + wi * W + rloc
                plsc.store_scatter(grow, [dest // GC, dest % GC],
                                   rabs, mask=m)
                plsc.store_scatter(gpos, [dest // GC, dest % GC],
                                   sv, mask=m)
            return u + n
        return lax.fori_loop(0, KPW, collect, u)
    u = lax.fori_loop(0, MAXWIN, collect_win, jnp.int32(0))

    @pl.when(u > 0)
    def _():
        r0 = jnp.full((L,), grow[0, pl.ds(0, L)][0], i32)
        p0 = jnp.full((L,), gpos[0, pl.ds(0, L)][0], i32)
        ulim = ((u + GC - 1) // GC) * GC
        for k in range(8):
            dest = u + k * L + lanes
            mk = dest < ulim
            plsc.store_scatter(grow, [dest // GC, dest % GC], r0, mask=mk)
            plsc.store_scatter(gpos, [dest // GC, dest % GC], p0, mask=mk)

        mb = wbuf0.at[pl.ds(0, GC)]
        vb = wbuf1.at[pl.ds(0, GC)]

        def chunk_c(g, _):
            pltpu.async_copy(mem_hbm.at[grow.at[g]], mb, gsem0)
            pltpu.async_copy(val_hbm.at[gpos.at[g]], vb, gsem1)
            pltpu.make_async_copy(mem_hbm.at[pl.ds(0, GC)], mb, gsem0).wait()
            pltpu.make_async_copy(val_hbm.at[pl.ds(0, GC)], vb, gsem1).wait()

            def blend(r, _):
                for t_ in range(D // L):
                    a = wbuf0[r, pl.ds(t_ * L, L)]
                    b = wbuf1[r, pl.ds(t_ * L, L)]
                    wbuf0[r, pl.ds(t_ * L, L)] = (
                        f32(1.0 - MOM) * a + f32(MOM) * b)
                return 0
            lax.fori_loop(0, GC, blend, 0)

            pltpu.async_copy(mb, out_hbm.at[grow.at[g]], gsem0)
            pltpu.make_async_copy(mb, out_hbm.at[pl.ds(0, GC)], gsem0).wait()
            return 0
        lax.fori_loop(0, (u + GC - 1) // GC, chunk_c, 0)


@jax.jit
def kernel(mem, val, idx):
    mesh = plsc.VectorSubcoreMesh(
        core_axis_name="c", subcore_axis_name="s",
        num_cores=NC, num_subcores=NS)
    f = pl.kernel(
        _body,
        out_type=jax.ShapeDtypeStruct((M, D), jnp.float32),
        mesh=mesh,
        compiler_params=pltpu.CompilerParams(needs_layout_passes=False),
        scratch_types=[
            pltpu.VMEM((IDXC,), jnp.int32),
            pltpu.VMEM((IDXC,), jnp.int32),
            pltpu.VMEM((IDXC + L,), jnp.int32),
            pltpu.VMEM((IDXC + L,), jnp.int32),
            pltpu.VMEM((SLOT_SZ,), jnp.int32),
            pltpu.VMEM((GROWS, GC), jnp.int32),
            pltpu.VMEM((GROWS, GC), jnp.int32),
            pltpu.VMEM((W, D), jnp.float32),
            pltpu.VMEM((W, D), jnp.float32),
            pltpu.VMEM((W, D), jnp.float32),
            pltpu.VMEM((W, D), jnp.float32),
            pltpu.SemaphoreType.DMA,
            pltpu.SemaphoreType.DMA,
            pltpu.SemaphoreType.DMA,
            pltpu.SemaphoreType.DMA,
            pltpu.SemaphoreType.DMA,
            pltpu.SemaphoreType.DMA,
            pltpu.SemaphoreType.DMA,
            pltpu.SemaphoreType.DMA,
            pltpu.SemaphoreType.DMA,
            pltpu.SemaphoreType.DMA,
            pltpu.SemaphoreType.DMA,
            pltpu.SemaphoreType.DMA,
        ],
    )
    return f(mem, val, idx)


# speedup vs baseline: 1.0894x; 1.0337x over previous
import jax
import jax.numpy as jnp
from jax import lax
from jax.experimental import pallas as pl
from jax.experimental.pallas import tpu as pltpu
from jax.experimental.pallas import tpu_sc as plsc

M = 100000
B = 16384
D = 128
MOM = 0.8

NC, NS, L = 2, 16, 16
NW = NC * NS
W = 200
TOTWIN = M // W
NBASE = TOTWIN // NW
EXTRA = TOTWIN % NW
MAXWIN = NBASE + 1
IDXC = 1024
NCHUNK = B // IDXC
KPW = (W + L - 1) // L
SSTR = KPW * L
SLOT_SZ = MAXWIN * SSTR + L
GC = 128
GROWS = (MAXWIN * W + GC - 1) // GC


_GATHER_DNUMS = lax.GatherDimensionNumbers(
    offset_dims=(), collapsed_slice_dims=(0,), start_index_map=(0,))


def _gather16(x, idx):
    return lax.gather(x, idx[:, None], _GATHER_DNUMS, slice_sizes=(1,),
                      mode=lax.GatherScatterMode.PROMISE_IN_BOUNDS)


def _prefix_incl(x, lanes):
    c = x
    for s in (1, 2, 4, 8):
        sh = _gather16(c, jnp.maximum(lanes - s, 0))
        c = c + jnp.where(lanes >= s, sh, jnp.int32(0))
    return c


def _body(mem_hbm, val_hbm, idx_hbm, out_hbm,
          ibuf0, ibuf1, rowl, posl, slot, grow, gpos,
          wbuf0, wbuf1, wbuf2, wbuf3,
          isem0, isem1, lsem0, lsem1, lsem2, lsem3,
          ssem0, ssem1, ssem2, ssem3, gsem0, gsem1):
    i32 = jnp.int32
    f32 = jnp.float32
    wid = lax.axis_index("s") * NC + lax.axis_index("c")
    nwin = jnp.where(wid < EXTRA, NBASE + 1, NBASE)
    first_win = wid * (NBASE + 1) - jnp.maximum(0, wid - EXTRA)
    lo = first_win * W
    rng = nwin * W
    lanes = lax.iota(i32, L)
    neg1 = jnp.full((L,), -1, i32)

    def init_slot(i, _):
        slot[pl.ds(i * L, L)] = neg1
        return 0
    lax.fori_loop(0, SLOT_SZ // L, init_slot, 0)

    def process_chunk(c, ibuf):
        def scan_vreg(i, off):
            v = ibuf[pl.ds(i * L, L)]
            m = (v >= lo) & (v < lo + rng)
            n = plsc.all_reduce_population_count(m)[0]

            @pl.when(n > 0)
            def _():
                mi = jnp.where(m, jnp.int32(1), jnp.int32(0))
                csum = _prefix_incl(mi, lanes)
                dest = off + csum - mi
                rl = v - lo
                widx = (rl // W) * SSTR + rl % W
                pos = c * IDXC + i * L + lanes
                plsc.store_scatter(rowl, [dest], widx, mask=m)
                plsc.store_scatter(posl, [dest], pos, mask=m)
            return off + n
        cnt = lax.fori_loop(0, IDXC // L, scan_vreg, jnp.int32(0))

        def dedup(i, _):
            rv = rowl[pl.ds(i * L, L)]
            pv = posl[pl.ds(i * L, L)]
            for l in range(L):
                mk = (lanes == l) & (i * L + l < cnt)
                plsc.store_scatter(slot, [rv], pv, mask=mk)
            return 0
        lax.fori_loop(0, (cnt + L - 1) // L, dedup, 0)

    def idx_load(c, ibuf, sem):
        pltpu.async_copy(idx_hbm.at[pl.ds(c * IDXC, IDXC)], ibuf, sem)

    def idx_wait(ibuf, sem):
        pltpu.make_async_copy(idx_hbm.at[pl.ds(0, IDXC)], ibuf, sem).wait()

    def load_issue(c, wb, sem):
        pltpu.async_copy(mem_hbm.at[pl.ds(lo + c * W, W)], wb, sem)

    def load_wait(wb, sem):
        pltpu.make_async_copy(mem_hbm.at[pl.ds(0, W)], wb, sem).wait()

    def store_issue(c, wb, sem):
        pltpu.async_copy(wb, out_hbm.at[pl.ds(lo + c * W, W)], sem)

    def store_wait(wb, sem):
        pltpu.make_async_copy(wb, out_hbm.at[pl.ds(0, W)], sem).wait()

    WB = ((wbuf0, lsem0, ssem0), (wbuf1, lsem1, ssem1),
          (wbuf2, lsem2, ssem2), (wbuf3, lsem3, ssem3))

    def win_step(c, s):
        wb_p, lsem_p, ssem_p = WB[s]
        wb_q, lsem_q, ssem_q = WB[(s + 3) % 4]

        @pl.when(c < nwin)
        def _():
            load_wait(wb_p, lsem_p)
            store_issue(c, wb_p, ssem_p)

        @pl.when(c + 3 < nwin)
        def _():
            @pl.when(c >= 1)
            def _():
                store_wait(wb_q, ssem_q)
            load_issue(c + 3, wb_q, lsem_q)

    idx_load(0, ibuf0, isem0)
    load_issue(0, wbuf0, lsem0)
    load_issue(1, wbuf1, lsem1)
    load_issue(2, wbuf2, lsem2)

    def chunk_quad(q, _):
        for s in range(4):
            c = 4 * q + s
            ib, isem = (ibuf0, isem0) if s % 2 == 0 else (ibuf1, isem1)
            nb, nsem = (ibuf1, isem1) if s % 2 == 0 else (ibuf0, isem0)
            win_step(c, s)
            idx_wait(ib, isem)
            if s < 3:
                idx_load(c + 1, nb, nsem)
            else:
                @pl.when(q < NCHUNK // 4 - 1)
                def _():
                    idx_load(c + 1, nb, nsem)
            process_chunk(c, ib)
        return 0
    lax.fori_loop(0, NCHUNK // 4, chunk_quad, 0)

    store_wait(wbuf0, ssem0)
    store_wait(wbuf1, ssem1)
    store_wait(wbuf2, ssem2)
    store_wait(wbuf3, ssem3)

    def collect_win(wi, u):
        def collect(k, u):
            rloc = k * L + lanes
            sv = slot[pl.ds(wi * SSTR + k * L, L)]
            m = (rloc < W) & (sv >= 0)
            n = plsc.all_reduce_population_count(m)[0]

            @pl.when(n > 0)
            def _():
                mi = jnp.where(m, jnp.int32(1), jnp.int32(0))
                csum = _prefix_incl(mi, lanes)
                dest = u + csum - mi
                rabs = lo + wi * W + rloc
                plsc.store_scatter(grow, [dest // GC, dest % GC],
                                   rabs, mask=m)
                plsc.store_scatter(gpos, [dest // GC, dest % GC],
                                   sv, mask=m)
            return u + n
        return lax.fori_loop(0, KPW, collect, u)
    u = lax.fori_loop(0, MAXWIN, collect_win, jnp.int32(0))

    @pl.when(u > 0)
    def _():
        r0 = jnp.full((L,), grow[0, pl.ds(0, L)][0], i32)
        p0 = jnp.full((L,), gpos[0, pl.ds(0, L)][0], i32)
        ulim = ((u + GC - 1) // GC) * GC
        for k in range(8):
            dest = u + k * L + lanes
            mk = dest < ulim
            plsc.store_scatter(grow, [dest // GC, dest % GC], r0, mask=mk)
            plsc.store_scatter(gpos, [dest // GC, dest % GC], p0, mask=mk)

        nch = (u + GC - 1) // GC
        CB = ((wbuf0, wbuf1, gsem0, gsem1, ssem0),
              (wbuf2, wbuf3, lsem0, lsem1, ssem1))

        def cg_issue(g, mwb, vwb, msem, vsem):
            pltpu.async_copy(mem_hbm.at[grow.at[g]],
                             mwb.at[pl.ds(0, GC)], msem)
            pltpu.async_copy(val_hbm.at[gpos.at[g]],
                             vwb.at[pl.ds(0, GC)], vsem)

        def cg_wait(mwb, vwb, msem, vsem):
            pltpu.make_async_copy(mem_hbm.at[pl.ds(0, GC)],
                                  mwb.at[pl.ds(0, GC)], msem).wait()
            pltpu.make_async_copy(val_hbm.at[pl.ds(0, GC)],
                                  vwb.at[pl.ds(0, GC)], vsem).wait()

        def sc_wait(mwb, scsem):
            pltpu.make_async_copy(mwb.at[pl.ds(0, GC)],
                                  out_hbm.at[pl.ds(0, GC)], scsem).wait()

        def chunk_c(g, mwb, vwb, msem, vsem, scsem, qm, qv, qms, qvs, qss):
            @pl.when(g < nch)
            def _():
                @pl.when(g + 1 < nch)
                def _():
                    @pl.when(g >= 1)
                    def _():
                        sc_wait(qm, qss)
                    cg_issue(g + 1, qm, qv, qms, qvs)
                cg_wait(mwb, vwb, msem, vsem)

                def blend(r, _):
                    for t_ in range(D // L):
                        a = mwb[r, pl.ds(t_ * L, L)]
                        b = vwb[r, pl.ds(t_ * L, L)]
                        mwb[r, pl.ds(t_ * L, L)] = (
                            f32(1.0 - MOM) * a + f32(MOM) * b)
                    return 0
                lax.fori_loop(0, GC, blend, 0)

                pltpu.async_copy(mwb.at[pl.ds(0, GC)],
                                 out_hbm.at[grow.at[g]], scsem)

        cg_issue(0, *CB[0][:4])

        def chunk_c_pair(p, _):
            chunk_c(2 * p, *CB[0], *CB[1])
            chunk_c(2 * p + 1, *CB[1], *CB[0])
            return 0
        lax.fori_loop(0, (nch + 1) // 2, chunk_c_pair, 0)

        @pl.when(nch % 2 == 1)
        def _():
            sc_wait(CB[0][0], CB[0][4])

        @pl.when(nch % 2 == 0)
        def _():
            sc_wait(CB[1][0], CB[1][4])


@jax.jit
def kernel(mem, val, idx):
    mesh = plsc.VectorSubcoreMesh(
        core_axis_name="c", subcore_axis_name="s",
        num_cores=NC, num_subcores=NS)
    f = pl.kernel(
        _body,
        out_type=jax.ShapeDtypeStruct((M, D), jnp.float32),
        mesh=mesh,
        compiler_params=pltpu.CompilerParams(needs_layout_passes=False),
        scratch_types=[
            pltpu.VMEM((IDXC,), jnp.int32),
            pltpu.VMEM((IDXC,), jnp.int32),
            pltpu.VMEM((IDXC + L,), jnp.int32),
            pltpu.VMEM((IDXC + L,), jnp.int32),
            pltpu.VMEM((SLOT_SZ,), jnp.int32),
            pltpu.VMEM((GROWS, GC), jnp.int32),
            pltpu.VMEM((GROWS, GC), jnp.int32),
            pltpu.VMEM((W, D), jnp.float32),
            pltpu.VMEM((W, D), jnp.float32),
            pltpu.VMEM((W, D), jnp.float32),
            pltpu.VMEM((W, D), jnp.float32),
            pltpu.SemaphoreType.DMA,
            pltpu.SemaphoreType.DMA,
            pltpu.SemaphoreType.DMA,
            pltpu.SemaphoreType.DMA,
            pltpu.SemaphoreType.DMA,
            pltpu.SemaphoreType.DMA,
            pltpu.SemaphoreType.DMA,
            pltpu.SemaphoreType.DMA,
            pltpu.SemaphoreType.DMA,
            pltpu.SemaphoreType.DMA,
            pltpu.SemaphoreType.DMA,
            pltpu.SemaphoreType.DMA,
        ],
    )
    return f(mem, val, idx)
